# baseline (speedup 1.0000x reference)
import functools

import jax
import jax.numpy as jnp
from jax import lax
from jax.experimental import pallas as pl
from jax.experimental.pallas import tpu as pltpu
from jax.experimental.pallas import tpu_sc as plsc

B = 16384
FEAT = 40960
HID = 512

NC = 2
NS = 16
NW = NC * NS

CHUNKS = (8192, 6144, 2048)
CHUNK = 64
NBUF = 3

BS = 2048


def _make_gather_body(bch, off):
    rows_per_j = bch // NS
    n_chunks = rows_per_j // CHUNK

    def _gather_body(stm_hbm, nstm_hbm, emb_hbm, out_hbm, idx_v, bufs,
                     gs0, gs1, gs2, ws0, ws1, ws2):
        wid = lax.axis_index("s") * NC + lax.axis_index("c")
        gsems = (gs0, gs1, gs2)
        wsems = (ws0, ws1, ws2)

        def run(idx_hbm, t, j):
            base = off + j * rows_per_j
            pltpu.sync_copy(idx_hbm.at[pl.ds(base, rows_per_j)], idx_v)

            def start_gather(i):
                b = i % NBUF
                return pltpu.async_copy(
                    emb_hbm.at[idx_v.at[pl.ds(i * CHUNK, CHUNK)]], bufs.at[b],
                    gsems[b])

            gh = {}
            wh = {}
            for k in range(min(NBUF - 1, n_chunks)):
                gh[k] = start_gather(k)
            for i in range(n_chunks):
                b = i % NBUF
                gh[i].wait()
                wh[i] = pltpu.async_copy(
                    bufs.at[b],
                    out_hbm.at[t, pl.ds(j * rows_per_j + i * CHUNK, CHUNK)],
                    wsems[b])
                n = i + NBUF - 1
                if n < n_chunks:
                    if n - NBUF >= 0:
                        wh.pop(n - NBUF).wait()
                    gh[n] = start_gather(n)
            for i in sorted(wh):
                wh[i].wait()

        @pl.when(wid < NS)
        def _():
            run(stm_hbm, 0, wid)

        @pl.when(wid >= NS)
        def _():
            run(nstm_hbm, 1, wid - NS)

    return _gather_body


@functools.lru_cache(maxsize=None)
def _get_sc_gather(bch, off):
    rows_per_j = bch // NS
    return pl.kernel(
        _make_gather_body(bch, off),
        out_type=jax.ShapeDtypeStruct((2, bch, HID), jnp.float32),
        mesh=plsc.VectorSubcoreMesh(
            core_axis_name="c", subcore_axis_name="s",
            num_cores=NC, num_subcores=NS,
        ),
        scratch_types=[
            pltpu.VMEM((rows_per_j,), jnp.int32),
            pltpu.VMEM((NBUF, CHUNK, HID), jnp.float32),
            pltpu.SemaphoreType.DMA,
            pltpu.SemaphoreType.DMA,
            pltpu.SemaphoreType.DMA,
            pltpu.SemaphoreType.DMA,
            pltpu.SemaphoreType.DMA,
            pltpu.SemaphoreType.DMA,
        ],
    )


def _mlp_body(g_ref, w1a_ref, w1b_ref, b1_ref, w2_ref, b2_ref, w3_ref, b3_ref,
              out_ref):
    stm = jnp.clip(g_ref[0], 0.0, 1.0)
    nstm = jnp.clip(g_ref[1], 0.0, 1.0)
    h = jnp.dot(stm, w1a_ref[...], preferred_element_type=jnp.float32)
    h = h + jnp.dot(nstm, w1b_ref[...], preferred_element_type=jnp.float32)
    h = jnp.clip(h + b1_ref[0], 0.0, 1.0)
    h = jnp.clip(
        jnp.dot(h, w2_ref[...], preferred_element_type=jnp.float32) + b2_ref[0],
        0.0, 1.0)
    out_ref[...] = jnp.tanh(
        jnp.dot(h, w3_ref[...], preferred_element_type=jnp.float32) + b3_ref[0])


def _mlp(g3, W1a, W1b, b1, W2, b2, W3, b3):
    bch = g3.shape[1]
    return pl.pallas_call(
        _mlp_body,
        grid=(bch // BS,),
        in_specs=[
            pl.BlockSpec((2, BS, HID), lambda i: (0, i, 0)),
            pl.BlockSpec((HID, 128), lambda i: (0, 0)),
            pl.BlockSpec((HID, 128), lambda i: (0, 0)),
            pl.BlockSpec((1, 128), lambda i: (0, 0)),
            pl.BlockSpec((128, 32), lambda i: (0, 0)),
            pl.BlockSpec((1, 32), lambda i: (0, 0)),
            pl.BlockSpec((32, 1), lambda i: (0, 0)),
            pl.BlockSpec((1, 1), lambda i: (0, 0)),
        ],
        out_specs=pl.BlockSpec((BS, 1), lambda i: (i, 0)),
        out_shape=jax.ShapeDtypeStruct((bch, 1), jnp.float32),
        compiler_params=pltpu.CompilerParams(
            dimension_semantics=("arbitrary",)),
    )(g3, W1a, W1b, b1, W2, b2, W3, b3)


def kernel(stm_idx, stm_off, nstm_idx, nstm_off, emb, W1, b1, W2, b2, W3, b3):
    W1a = W1[:HID]
    W1b = W1[HID:]
    b1r = b1.reshape(1, 128)
    b2r = b2.reshape(1, 32)
    b3r = b3.reshape(1, 1)
    outs = []
    off = 0
    for bch in CHUNKS:
        g = _get_sc_gather(bch, off)(stm_idx, nstm_idx, emb)
        outs.append(_mlp(g, W1a, W1b, b1r, W2, b2r, W3, b3r))
        off += bch
    return jnp.concatenate(outs, axis=0)


# speedup vs baseline: 1.0208x; 1.0208x over previous
import functools

import jax
import jax.numpy as jnp
from jax import lax
from jax.experimental import pallas as pl
from jax.experimental.pallas import tpu as pltpu
from jax.experimental.pallas import tpu_sc as plsc

B = 16384
FEAT = 40960
HID = 512

NC = 2
NS = 16
NW = NC * NS

CHUNKS = (8192, 8192)
CHUNK = 64
NBUF = 3

BS = 2048


def _make_gather_body(bch, off):
    rows_per_j = bch // NS
    n_chunks = rows_per_j // CHUNK

    def _gather_body(stm_hbm, nstm_hbm, emb_hbm, out_hbm, idx_v, bufs,
                     gs0, gs1, gs2, ws0, ws1, ws2):
        wid = lax.axis_index("s") * NC + lax.axis_index("c")
        gsems = (gs0, gs1, gs2)
        wsems = (ws0, ws1, ws2)

        def run(idx_hbm, t, j):
            base = off + j * rows_per_j
            pltpu.sync_copy(idx_hbm.at[pl.ds(base, rows_per_j)], idx_v)

            def start_gather(i):
                b = i % NBUF
                return pltpu.async_copy(
                    emb_hbm.at[idx_v.at[pl.ds(i * CHUNK, CHUNK)]], bufs.at[b],
                    gsems[b])

            gh = {}
            wh = {}
            for k in range(min(NBUF - 1, n_chunks)):
                gh[k] = start_gather(k)
            for i in range(n_chunks):
                b = i % NBUF
                gh[i].wait()
                wh[i] = pltpu.async_copy(
                    bufs.at[b],
                    out_hbm.at[t, pl.ds(j * rows_per_j + i * CHUNK, CHUNK)],
                    wsems[b])
                n = i + NBUF - 1
                if n < n_chunks:
                    if n - NBUF >= 0:
                        wh.pop(n - NBUF).wait()
                    gh[n] = start_gather(n)
            for i in sorted(wh):
                wh[i].wait()

        @pl.when(wid < NS)
        def _():
            run(stm_hbm, 0, wid)

        @pl.when(wid >= NS)
        def _():
            run(nstm_hbm, 1, wid - NS)

    return _gather_body


@functools.lru_cache(maxsize=None)
def _get_sc_gather(bch, off):
    rows_per_j = bch // NS
    return pl.kernel(
        _make_gather_body(bch, off),
        out_type=jax.ShapeDtypeStruct((2, bch, HID), jnp.float32),
        mesh=plsc.VectorSubcoreMesh(
            core_axis_name="c", subcore_axis_name="s",
            num_cores=NC, num_subcores=NS,
        ),
        scratch_types=[
            pltpu.VMEM((rows_per_j,), jnp.int32),
            pltpu.VMEM((NBUF, CHUNK, HID), jnp.float32),
            pltpu.SemaphoreType.DMA,
            pltpu.SemaphoreType.DMA,
            pltpu.SemaphoreType.DMA,
            pltpu.SemaphoreType.DMA,
            pltpu.SemaphoreType.DMA,
            pltpu.SemaphoreType.DMA,
        ],
    )


def _mlp_body(g_ref, w1a_ref, w1b_ref, b1_ref, w2_ref, b2_ref, w3_ref, b3_ref,
              out_ref):
    stm = jnp.clip(g_ref[0], 0.0, 1.0)
    nstm = jnp.clip(g_ref[1], 0.0, 1.0)
    h = jnp.dot(stm, w1a_ref[...], preferred_element_type=jnp.float32)
    h = h + jnp.dot(nstm, w1b_ref[...], preferred_element_type=jnp.float32)
    h = jnp.clip(h + b1_ref[0], 0.0, 1.0)
    h = jnp.clip(
        jnp.dot(h, w2_ref[...], preferred_element_type=jnp.float32) + b2_ref[0],
        0.0, 1.0)
    out_ref[...] = jnp.tanh(
        jnp.dot(h, w3_ref[...], preferred_element_type=jnp.float32) + b3_ref[0])


def _mlp(g3, W1a, W1b, b1, W2, b2, W3, b3):
    bch = g3.shape[1]
    return pl.pallas_call(
        _mlp_body,
        grid=(bch // BS,),
        in_specs=[
            pl.BlockSpec((2, BS, HID), lambda i: (0, i, 0)),
            pl.BlockSpec((HID, 128), lambda i: (0, 0)),
            pl.BlockSpec((HID, 128), lambda i: (0, 0)),
            pl.BlockSpec((1, 128), lambda i: (0, 0)),
            pl.BlockSpec((128, 32), lambda i: (0, 0)),
            pl.BlockSpec((1, 32), lambda i: (0, 0)),
            pl.BlockSpec((32, 1), lambda i: (0, 0)),
            pl.BlockSpec((1, 1), lambda i: (0, 0)),
        ],
        out_specs=pl.BlockSpec((BS, 1), lambda i: (i, 0)),
        out_shape=jax.ShapeDtypeStruct((bch, 1), jnp.float32),
        compiler_params=pltpu.CompilerParams(
            dimension_semantics=("arbitrary",)),
    )(g3, W1a, W1b, b1, W2, b2, W3, b3)


def kernel(stm_idx, stm_off, nstm_idx, nstm_off, emb, W1, b1, W2, b2, W3, b3):
    W1a = W1[:HID]
    W1b = W1[HID:]
    b1r = b1.reshape(1, 128)
    b2r = b2.reshape(1, 32)
    b3r = b3.reshape(1, 1)
    outs = []
    off = 0
    for bch in CHUNKS:
        g = _get_sc_gather(bch, off)(stm_idx, nstm_idx, emb)
        outs.append(_mlp(g, W1a, W1b, b1r, W2, b2r, W3, b3r))
        off += bch
    return jnp.concatenate(outs, axis=0)
